# baseline (speedup 1.0000x reference)
import functools

import jax
import jax.numpy as jnp
from jax import lax
from jax.experimental import pallas as pl
from jax.experimental.pallas import tpu as pltpu
from jax.experimental.pallas import tpu_sc as plsc

_LANES = 16


def _k_split(n_nodes: int):
    nk = 5
    kc_len = -(-n_nodes // (nk * 128)) * 128
    return nk, kc_len


def _make_sc_histogram(n_nodes: int, n_edges: int):
    info = plsc.get_sparse_core_info()
    nc, ns = info.num_cores, info.num_subcores
    nw = nc * ns
    assert n_edges % (nw * _LANES) == 0
    blk = 128
    nblocks = n_edges // blk
    bpw = nblocks // nw
    extra = nblocks - bpw * nw
    dma_blocks = bpw + (1 if extra else 0)
    dma_len = dma_blocks * blk
    switch_tile = nw - extra

    nk, kc_len = _k_split(n_nodes)
    hist_len = nk * kc_len

    mesh = plsc.VectorSubcoreMesh(core_axis_name="c", subcore_axis_name="s")

    @functools.partial(
        pl.kernel,
        mesh=mesh,
        out_type=jax.ShapeDtypeStruct((nk * nw, kc_len), jnp.float32),
        scratch_types=[
            pltpu.VMEM((2, dma_len), jnp.int32),
            pltpu.VMEM((dma_len,), jnp.float32),
            pltpu.VMEM((hist_len,), jnp.float32),
        ],
        compiler_params=pltpu.CompilerParams(needs_layout_passes=False),
    )
    def hist_kernel(ei_hbm, w_hbm, out_hbm, idx_v, w_v, hist_v):
        wid = lax.axis_index("s") * nc + lax.axis_index("c")
        start_blk = wid * bpw + jnp.maximum(wid - switch_tile, 0)
        base = start_blk * blk
        pltpu.sync_copy(ei_hbm.at[:, pl.ds(base, dma_len)], idx_v)
        pltpu.sync_copy(w_hbm.at[pl.ds(base, dma_len)], w_v)

        zero = jnp.zeros((_LANES,), jnp.float32)

        def zbody(i, _):
            hist_v[pl.ds(i * _LANES, _LANES)] = zero
            return 0

        lax.fori_loop(0, hist_len // _LANES, zbody, 0, unroll=8)

        def body(i, _):
            off = i * _LANES
            idx = idx_v[0, pl.ds(off, _LANES)]
            w = w_v[pl.ds(off, _LANES)]
            plsc.addupdate_scatter(hist_v, [idx], w)
            return 0

        lax.fori_loop(0, bpw * blk // _LANES, body, 0, unroll=8)

        if extra:
            @pl.when(wid >= switch_tile)
            def _():
                lax.fori_loop(
                    bpw * blk // _LANES, dma_blocks * blk // _LANES, body, 0,
                    unroll=8,
                )

        for j in range(nk):
            pltpu.sync_copy(
                hist_v.at[pl.ds(j * kc_len, kc_len)], out_hbm.at[j * nw + wid]
            )

    return hist_kernel


def _make_dense(n_nodes: int, n_feat: int, n_hid: int, nw: int):
    nk, kc_len = _k_split(n_nodes)

    def dense_body(cp_ref, x_ref, w1_ref, b1_ref, w2t_ref, b2_ref, out_ref,
                   acc_ref):
        kc = pl.program_id(0)

        @pl.when(kc == 0)
        def _():
            acc_ref[...] = jnp.zeros_like(acc_ref)

        row = jax.lax.broadcasted_iota(jnp.int32, (kc_len, n_feat), 0)
        xv = jnp.where(row + kc * kc_len < n_nodes, x_ref[...], 0.0)
        acc_ref[...] += jnp.dot(
            cp_ref[...], xv, preferred_element_type=jnp.float32,
            precision=jax.lax.Precision.HIGHEST)

        @pl.when(kc == nk - 1)
        def _():
            pooled = jnp.sum(acc_ref[...], axis=0, keepdims=True) * (1.0 / n_nodes)
            pooled = jnp.maximum(pooled, 0.0)
            z = jnp.sum(pooled * w1_ref[...], axis=1, keepdims=True) + b1_ref[...]
            z = jnp.maximum(z, 0.0)
            out_ref[...] = (jnp.sum(z * w2t_ref[...], axis=0, keepdims=True)
                            + b2_ref[...])

    return pl.pallas_call(
        dense_body,
        grid=(nk,),
        in_specs=[
            pl.BlockSpec((nw, kc_len), lambda kc: (kc, 0)),
            pl.BlockSpec((kc_len, n_feat), lambda kc: (kc, 0)),
            pl.BlockSpec((n_hid, n_feat), lambda kc: (0, 0)),
            pl.BlockSpec((n_hid, 1), lambda kc: (0, 0)),
            pl.BlockSpec((n_hid, 1), lambda kc: (0, 0)),
            pl.BlockSpec((1, 1), lambda kc: (0, 0)),
        ],
        out_specs=pl.BlockSpec((1, 1), lambda kc: (0, 0)),
        out_shape=jax.ShapeDtypeStruct((1, 1), jnp.float32),
        scratch_shapes=[pltpu.VMEM((nw, n_feat), jnp.float32)],
    )


def kernel(x, edge_index, edge_attr, W1, b1, W2, b2):
    n_nodes, n_feat = x.shape
    n_edges = edge_attr.shape[0]
    n_hid = W1.shape[0]

    cp = _make_sc_histogram(n_nodes, n_edges)(edge_index, edge_attr)

    nk, _ = _k_split(n_nodes)
    out = _make_dense(n_nodes, n_feat, n_hid, cp.shape[0] // nk)(
        cp, x, W1, b1.reshape(-1, 1), W2.reshape(-1, 1), b2.reshape(1, 1))
    return out


# speedup vs baseline: 1.0318x; 1.0318x over previous
import functools

import jax
import jax.numpy as jnp
from jax import lax
from jax.experimental import pallas as pl
from jax.experimental.pallas import tpu as pltpu
from jax.experimental.pallas import tpu_sc as plsc

_LANES = 16


def _make_sc_histogram(n_nodes: int, n_edges: int):
    info = plsc.get_sparse_core_info()
    nc, ns = info.num_cores, info.num_subcores
    nw = nc * ns
    assert n_edges % (nw * _LANES) == 0
    blk = 128
    nblocks = n_edges // blk
    bpw = nblocks // nw
    extra = nblocks - bpw * nw
    dma_blocks = bpw + (1 if extra else 0)
    dma_len = dma_blocks * blk
    switch_tile = nw - extra

    mesh = plsc.VectorSubcoreMesh(core_axis_name="c", subcore_axis_name="s")

    @functools.partial(
        pl.kernel,
        mesh=mesh,
        out_type=jax.ShapeDtypeStruct((nw, n_nodes), jnp.float32),
        scratch_types=[
            pltpu.VMEM((2, dma_len), jnp.int32),
            pltpu.VMEM((dma_len,), jnp.float32),
            pltpu.VMEM((n_nodes,), jnp.float32),
        ],
        compiler_params=pltpu.CompilerParams(needs_layout_passes=False),
    )
    def hist_kernel(ei_hbm, w_hbm, out_hbm, idx_v, w_v, hist_v):
        wid = lax.axis_index("s") * nc + lax.axis_index("c")
        start_blk = wid * bpw + jnp.maximum(wid - switch_tile, 0)
        base = start_blk * blk
        pltpu.sync_copy(ei_hbm.at[:, pl.ds(base, dma_len)], idx_v)
        pltpu.sync_copy(w_hbm.at[pl.ds(base, dma_len)], w_v)

        zero = jnp.zeros((_LANES,), jnp.float32)

        def zbody(i, _):
            hist_v[pl.ds(i * _LANES, _LANES)] = zero
            return 0

        lax.fori_loop(0, n_nodes // _LANES, zbody, 0, unroll=8)

        def body(i, _):
            off = i * _LANES
            idx = idx_v[0, pl.ds(off, _LANES)]
            w = w_v[pl.ds(off, _LANES)]
            plsc.addupdate_scatter(hist_v, [idx], w)
            return 0

        lax.fori_loop(0, bpw * blk // _LANES, body, 0, unroll=8)

        if extra:
            @pl.when(wid >= switch_tile)
            def _():
                lax.fori_loop(
                    bpw * blk // _LANES, dma_blocks * blk // _LANES, body, 0,
                    unroll=8,
                )

        pltpu.sync_copy(hist_v, out_hbm.at[wid])

    return hist_kernel


def _dense_body(cp_ref, x_ref, w1_ref, b1_ref, w2t_ref, b2_ref, out_ref):
    n_nodes = x_ref.shape[0]
    s = jnp.dot(cp_ref[...], x_ref[...], preferred_element_type=jnp.float32,
                precision=jax.lax.Precision.HIGHEST)
    pooled = jnp.sum(s, axis=0, keepdims=True) * (1.0 / n_nodes)
    pooled = jnp.maximum(pooled, 0.0)
    z = jnp.sum(pooled * w1_ref[...], axis=1, keepdims=True) + b1_ref[...]
    z = jnp.maximum(z, 0.0)
    out_ref[...] = jnp.sum(z * w2t_ref[...], axis=0, keepdims=True) + b2_ref[...]


def kernel(x, edge_index, edge_attr, W1, b1, W2, b2):
    n_nodes, _ = x.shape
    n_edges = edge_attr.shape[0]

    cp = _make_sc_histogram(n_nodes, n_edges)(edge_index, edge_attr)

    out = pl.pallas_call(
        _dense_body,
        out_shape=jax.ShapeDtypeStruct((1, 1), jnp.float32),
    )(cp, x, W1, b1.reshape(-1, 1), W2.reshape(-1, 1), b2.reshape(1, 1))
    return out
